# baseline (speedup 1.0000x reference)
import functools

import jax
import jax.numpy as jnp
from jax import lax
from jax.experimental import pallas as pl
from jax.experimental.pallas import tpu as pltpu
from jax.experimental.pallas import tpu_sc as plsc

B = 16384
NC, NS, L = 2, 16, 16
NW = NC * NS
BPW = B // NW
CHUNK = 128
NCHUNK = BPW // CHUNK


@functools.partial(
    pl.kernel,
    mesh=plsc.VectorSubcoreMesh(core_axis_name="c", subcore_axis_name="s"),
    out_type=jax.ShapeDtypeStruct((B,), jnp.float32),
    scratch_types=[
        pltpu.VMEM((BPW,), jnp.int32),
        pltpu.VMEM((BPW,), jnp.float32),
        pltpu.SemaphoreType.DMA,
        pltpu.SemaphoreType.DMA,
        pltpu.SemaphoreType.DMA,
        pltpu.SemaphoreType.DMA,
        pltpu.SemaphoreType.DMA,
    ],
)
def _gather_sigmoid(idx_hbm, alpha_hbm, out_hbm, idx_v, vals_v,
                    sem_i0, sem_i1, sem_g0, sem_g1, sem_o):
    wid = lax.axis_index("s") * NC + lax.axis_index("c")
    base = wid * BPW
    H = BPW // 2
    sem_i = (sem_i0, sem_i1)
    sem_g = (sem_g0, sem_g1)

    idx_cp = [
        pltpu.async_copy(idx_hbm.at[pl.ds(base + h * H, H)],
                         idx_v.at[pl.ds(h * H, H)], sem_i[h])
        for h in range(2)
    ]
    gathers = []
    for h in range(2):
        idx_cp[h].wait()
        gathers.append(
            pltpu.async_copy(alpha_hbm.at[idx_v.at[pl.ds(h * H, H)]],
                             vals_v.at[pl.ds(h * H, H)], sem_g[h]))
    one = jnp.full((L,), 1.0, dtype=jnp.float32)
    out_cp = []
    for h in range(2):
        gathers[h].wait()
        for i in range(h * H // L, (h + 1) * H // L):
            x = vals_v[pl.ds(i * L, L)]
            vals_v[pl.ds(i * L, L)] = one / (one + jnp.exp(-x))
        out_cp.append(
            pltpu.async_copy(vals_v.at[pl.ds(h * H, H)],
                             out_hbm.at[pl.ds(base + h * H, H)], sem_o))
    for c in out_cp:
        c.wait()


def kernel(idx, alpha):
    return _gather_sigmoid(idx.astype(jnp.int32), alpha)
